# Initial kernel scaffold; baseline (speedup 1.0000x reference)
import functools

import jax
import jax.numpy as jnp
from jax import lax
from jax.experimental import pallas as pl
from jax.experimental.pallas import tpu as pltpu
from jax.experimental.pallas import tpu_sc as plsc

BATCH = 4096
SEQ = 200
D = 64
N = BATCH * SEQ

_info = plsc.get_sparse_core_info()
_NC, _NS = _info.num_cores, _info.num_subcores
NW = _NC * _NS
B_PER_W = N // NW
CHUNK = 512
NCHUNK = B_PER_W // CHUNK

_mesh = plsc.VectorSubcoreMesh(core_axis_name="c", subcore_axis_name="s")


@functools.partial(
    pl.kernel,
    out_type=jax.ShapeDtypeStruct((N, D), jnp.float32),
    mesh=_mesh,
    scratch_types=[
        pltpu.VMEM((CHUNK,), jnp.int32),
        pltpu.VMEM((CHUNK, D), jnp.float32),
        pltpu.SemaphoreType.DMA,
    ],
)
def _embed(idx_hbm, table_hbm, out_hbm, idx_v, rows_v, sem):
    wid = lax.axis_index("s") * _NC + lax.axis_index("c")
    base = wid * B_PER_W

    def body(i, carry):
        off = base + i * CHUNK
        pltpu.sync_copy(idx_hbm.at[pl.ds(off, CHUNK)], idx_v)
        pltpu.async_copy(table_hbm.at[idx_v], rows_v, sem).wait()
        pltpu.sync_copy(rows_v, out_hbm.at[pl.ds(off, CHUNK)])
        return carry

    lax.fori_loop(0, NCHUNK, body, 0)


def kernel(word_sequences, embedding_weight):
    flat = word_sequences.reshape(N)
    out = _embed(flat, embedding_weight)
    return out.reshape(BATCH, SEQ, D)


# speedup vs baseline: 3.5786x; 3.5786x over previous
import functools

import jax
import jax.numpy as jnp
from jax import lax
from jax.experimental import pallas as pl
from jax.experimental.pallas import tpu as pltpu
from jax.experimental.pallas import tpu_sc as plsc

BATCH = 4096
SEQ = 200
D = 64
N = BATCH * SEQ

_info = plsc.get_sparse_core_info()
_NC, _NS = _info.num_cores, _info.num_subcores
NW = _NC * _NS
B_PER_W = N // NW
CHUNK = 512
NCHUNK = B_PER_W // CHUNK

_mesh = plsc.VectorSubcoreMesh(core_axis_name="c", subcore_axis_name="s")


@functools.partial(
    pl.kernel,
    out_type=jax.ShapeDtypeStruct((N, D), jnp.float32),
    mesh=_mesh,
    scratch_types=[
        pltpu.VMEM((CHUNK,), jnp.int32),
        pltpu.VMEM((CHUNK, D), jnp.float32),
        pltpu.SemaphoreType.DMA,
    ],
    compiler_params=pltpu.CompilerParams(use_tc_tiling_on_sc=False),
)
def _embed(idx_hbm, table_hbm, out_hbm, idx_v, rows_v, sem):
    wid = lax.axis_index("s") * _NC + lax.axis_index("c")
    base = wid * B_PER_W

    def body(i, carry):
        off = base + i * CHUNK
        pltpu.sync_copy(idx_hbm.at[pl.ds(off, CHUNK)], idx_v)
        pltpu.async_copy(table_hbm.at[idx_v], rows_v, sem).wait()
        pltpu.sync_copy(rows_v, out_hbm.at[pl.ds(off, CHUNK)])
        return carry

    lax.fori_loop(0, NCHUNK, body, 0)


def kernel(word_sequences, embedding_weight):
    flat = word_sequences.reshape(N)
    out = _embed(flat, embedding_weight)
    return out.reshape(BATCH, SEQ, D)


# speedup vs baseline: 3.5975x; 1.0053x over previous
import functools

import jax
import jax.numpy as jnp
from jax import lax
from jax.experimental import pallas as pl
from jax.experimental.pallas import tpu as pltpu
from jax.experimental.pallas import tpu_sc as plsc

BATCH = 4096
SEQ = 200
D = 64
N = BATCH * SEQ

_info = plsc.get_sparse_core_info()
_NC, _NS = _info.num_cores, _info.num_subcores
NW = _NC * _NS
B_PER_W = N // NW
CHUNK = 512
NCHUNK = B_PER_W // CHUNK

_mesh = plsc.VectorSubcoreMesh(core_axis_name="c", subcore_axis_name="s")


@functools.partial(
    pl.kernel,
    out_type=jax.ShapeDtypeStruct((N, D), jnp.float32),
    mesh=_mesh,
    scratch_types=[
        pltpu.VMEM((B_PER_W,), jnp.int32),
        pltpu.VMEM((CHUNK, D), jnp.float32),
        pltpu.VMEM((CHUNK, D), jnp.float32),
        pltpu.SemaphoreType.DMA,
        pltpu.SemaphoreType.DMA,
        pltpu.SemaphoreType.DMA,
        pltpu.SemaphoreType.DMA,
    ],
    compiler_params=pltpu.CompilerParams(use_tc_tiling_on_sc=False),
)
def _embed(idx_hbm, table_hbm, out_hbm, idx_v, rows0, rows1, sg0, sg1, sw0, sw1):
    wid = lax.axis_index("s") * _NC + lax.axis_index("c")
    base = wid * B_PER_W
    rows = (rows0, rows1)
    sg = (sg0, sg1)
    sw = (sw0, sw1)

    pltpu.sync_copy(idx_hbm.at[pl.ds(base, B_PER_W)], idx_v)

    def gather_start(i, b):
        pltpu.async_copy(
            table_hbm.at[idx_v.at[pl.ds(i * CHUNK, CHUNK)]], rows[b], sg[b]
        )

    def gather_wait(i, b):
        pltpu.make_async_copy(
            table_hbm.at[idx_v.at[pl.ds(i * CHUNK, CHUNK)]], rows[b], sg[b]
        ).wait()

    def wb_start(i, b):
        pltpu.async_copy(rows[b], out_hbm.at[pl.ds(base + i * CHUNK, CHUNK)], sw[b])

    def wb_wait(i, b):
        pltpu.make_async_copy(
            rows[b], out_hbm.at[pl.ds(base + i * CHUNK, CHUNK)], sw[b]
        ).wait()

    gather_start(0, 0)
    gather_start(1, 1)
    gather_wait(0, 0)
    wb_start(0, 0)
    gather_wait(1, 1)
    wb_start(1, 1)

    def body(g, carry):
        for b in range(2):
            i = 2 * g + b
            wb_wait(i - 2, b)
            gather_start(i, b)
            gather_wait(i, b)
            wb_start(i, b)
        return carry

    lax.fori_loop(1, NCHUNK // 2, body, 0)

    wb_wait(NCHUNK - 2, 0)
    wb_wait(NCHUNK - 1, 1)


def kernel(word_sequences, embedding_weight):
    flat = word_sequences.reshape(N)
    out = _embed(flat, embedding_weight)
    return out.reshape(BATCH, SEQ, D)


# speedup vs baseline: 4.9770x; 1.3835x over previous
import functools

import jax
import jax.numpy as jnp
from jax import lax
from jax.experimental import pallas as pl
from jax.experimental.pallas import tpu as pltpu
from jax.experimental.pallas import tpu_sc as plsc

BATCH = 4096
SEQ = 200
D = 64
N = BATCH * SEQ

_info = plsc.get_sparse_core_info()
_NC, _NS = _info.num_cores, _info.num_subcores
NW = _NC * _NS
B_PER_W = N // NW
CHUNK = 320
NCHUNK = B_PER_W // CHUNK

_mesh = plsc.VectorSubcoreMesh(core_axis_name="c", subcore_axis_name="s")


@functools.partial(
    pl.kernel,
    out_type=jax.ShapeDtypeStruct((N, D), jnp.float32),
    mesh=_mesh,
    scratch_types=[
        pltpu.VMEM((B_PER_W,), jnp.int32),
        pltpu.VMEM_SHARED((1001, D), jnp.float32),
        pltpu.VMEM((CHUNK, D), jnp.float32),
        pltpu.VMEM((CHUNK, D), jnp.float32),
        pltpu.SemaphoreType.DMA,
        pltpu.SemaphoreType.DMA,
        pltpu.SemaphoreType.DMA,
        pltpu.SemaphoreType.DMA,
    ],
    compiler_params=pltpu.CompilerParams(use_tc_tiling_on_sc=False),
)
def _embed(
    idx_hbm, table_hbm, out_hbm, idx_v, table_v, rows0, rows1, sg0, sg1, sw0, sw1
):
    wid = lax.axis_index("s") * _NC + lax.axis_index("c")
    base = wid * B_PER_W
    rows = (rows0, rows1)
    sg = (sg0, sg1)
    sw = (sw0, sw1)

    pltpu.sync_copy(idx_hbm.at[pl.ds(base, B_PER_W)], idx_v)

    @pl.when(lax.axis_index("s") == 0)
    def _load_table():
        pltpu.sync_copy(table_hbm, table_v)

    plsc.subcore_barrier()

    def gather_start(i, b):
        pltpu.async_copy(
            table_v.at[idx_v.at[pl.ds(i * CHUNK, CHUNK)]], rows[b], sg[b]
        )

    def gather_wait(i, b):
        pltpu.make_async_copy(
            table_v.at[idx_v.at[pl.ds(i * CHUNK, CHUNK)]], rows[b], sg[b]
        ).wait()

    def wb_start(i, b):
        pltpu.async_copy(rows[b], out_hbm.at[pl.ds(base + i * CHUNK, CHUNK)], sw[b])

    def wb_wait(i, b):
        pltpu.make_async_copy(
            rows[b], out_hbm.at[pl.ds(base + i * CHUNK, CHUNK)], sw[b]
        ).wait()

    gather_start(0, 0)
    gather_start(1, 1)
    gather_wait(0, 0)
    wb_start(0, 0)
    gather_wait(1, 1)
    wb_start(1, 1)

    def body(g, carry):
        for b in range(2):
            i = 2 * g + b
            wb_wait(i - 2, b)
            gather_start(i, b)
            gather_wait(i, b)
            wb_start(i, b)
        return carry

    lax.fori_loop(1, NCHUNK // 2, body, 0)

    wb_wait(NCHUNK - 2, 0)
    wb_wait(NCHUNK - 1, 1)


def kernel(word_sequences, embedding_weight):
    flat = word_sequences.reshape(N)
    out = _embed(flat, embedding_weight)
    return out.reshape(BATCH, SEQ, D)
